# baseline (speedup 1.0000x reference)
import functools

import jax
import jax.numpy as jnp
from jax import lax
from jax.experimental import pallas as pl
from jax.experimental.pallas import tpu as pltpu
from jax.experimental.pallas import tpu_sc as plsc

N_COLS = 26
VOCAB = 100
DIM = 50
BATCH = 16384
OUTC = N_COLS * DIM
OUTC_PAD = 1304
NTR = OUTC_PAD // 8

NC, NS, L = 2, 16, 16
NW = NC * NS
BBLK = BATCH // NW
NT = BBLK // 128
NV = BBLK // L
GB = 25


def _make_kernel():
    mesh = plsc.VectorSubcoreMesh(core_axis_name="c", subcore_axis_name="s")

    @functools.partial(
        pl.kernel,
        out_type=jax.ShapeDtypeStruct((NTR, BATCH // 128, 8, 128), jnp.float32),
        mesh=mesh,
        scratch_types=[
            pltpu.VMEM((2, BBLK), jnp.int32),
            pltpu.VMEM((2, DIM * VOCAB), jnp.float32),
            pltpu.VMEM((16, NT, 8, 128), jnp.float32),
            pltpu.SemaphoreType.DMA,
            pltpu.SemaphoreType.DMA,
        ],
        compiler_params=pltpu.CompilerParams(
            use_tc_tiling_on_sc=False, needs_layout_passes=False
        ),
    )
    def emb(cat_t_hbm, tab_t_hbm, out_hbm, idx_v, tab_v, out_v, sem_in, sem_out):
        w = lax.axis_index("s") * NC + lax.axis_index("c")
        b0 = w * BBLK
        t0 = w * NT

        def load_unit(i, slot):
            cp_i = pltpu.async_copy(
                cat_t_hbm.at[i, pl.ds(b0, BBLK)], idx_v.at[slot], sem_in
            )
            cp_t = pltpu.async_copy(tab_t_hbm.at[i], tab_v.at[slot], sem_in)
            return cp_i, cp_t

        def compute_unit(i, slot):
            phi = (DIM * i) % 8

            def vec_body(s, carry):
                t, sl = s // 8, s % 8
                iv = idx_v[slot, pl.ds(s * L, L)]
                for j0 in range(0, DIM, GB):
                    vals = [
                        plsc.load_gather(tab_v.at[slot], [iv + jl * VOCAB])
                        for jl in range(j0, j0 + GB)
                    ]
                    for jl, v in zip(range(j0, j0 + GB), vals):
                        al, sub = (phi + jl) // 8, (phi + jl) % 8
                        out_v[slot * 8 + al, t, sub, pl.ds(sl * L, L)] = v
                return carry

            lax.fori_loop(0, NV, vec_body, 0)

        def store_unit(i, slot):
            phi = (DIM * i) % 8
            a0 = (DIM * i) // 8
            span = (phi + DIM + 7) // 8
            copies = []
            for al in range(span):
                r0 = phi if al == 0 else 0
                r1 = phi + DIM - 8 * al
                r1 = 8 if r1 > 8 else r1
                rn = r1 - r0
                copies.append(
                    pltpu.async_copy(
                        out_v.at[slot * 8 + al, :, pl.ds(r0, rn), :],
                        out_hbm.at[a0 + al, pl.ds(t0, NT), pl.ds(r0, rn), :],
                        sem_out,
                    )
                )
            return copies

        loads = load_unit(0, 0)
        stores = [None, None]
        for i in range(N_COLS):
            slot = i % 2
            for cp in loads:
                cp.wait()
            if i + 1 < N_COLS:
                loads = load_unit(i + 1, (i + 1) % 2)
            if stores[slot] is not None:
                for cp in stores[slot]:
                    cp.wait()
                stores[slot] = None
            compute_unit(i, slot)
            stores[slot] = store_unit(i, slot)
        for st in stores:
            if st is not None:
                for cp in st:
                    cp.wait()

    return emb


_emb = _make_kernel()


def kernel(cat_tensor, tables):
    cat_t = cat_tensor.T
    tab_t = tables.transpose(0, 2, 1).reshape(N_COLS, DIM * VOCAB)
    out4 = _emb(cat_t, tab_t)
    out = out4.transpose(0, 2, 1, 3).reshape(OUTC_PAD, BATCH).T
    return out[:, :OUTC]


# speedup vs baseline: 1.0580x; 1.0580x over previous
import functools

import jax
import jax.numpy as jnp
from jax import lax
from jax.experimental import pallas as pl
from jax.experimental.pallas import tpu as pltpu
from jax.experimental.pallas import tpu_sc as plsc

N_COLS = 26
VOCAB = 100
DIM = 50
BATCH = 16384
OUTC = N_COLS * DIM
OUTC_PAD = 1304
NTR = OUTC_PAD // 8

NC, NS, L = 2, 16, 16
NW = NC * NS
BBLK = BATCH // NW
NT = BBLK // 128
NV = BBLK // L
GB = 10


def _make_kernel():
    mesh = plsc.VectorSubcoreMesh(core_axis_name="c", subcore_axis_name="s")

    @functools.partial(
        pl.kernel,
        out_type=jax.ShapeDtypeStruct((NTR, BATCH // 128, 8, 128), jnp.float32),
        mesh=mesh,
        scratch_types=[
            pltpu.VMEM((2, BBLK), jnp.int32),
            pltpu.VMEM((2, DIM * VOCAB), jnp.float32),
            pltpu.VMEM((16, NT, 8, 128), jnp.float32),
            pltpu.SemaphoreType.DMA,
            pltpu.SemaphoreType.DMA,
        ],
        compiler_params=pltpu.CompilerParams(
            use_tc_tiling_on_sc=False, needs_layout_passes=False
        ),
    )
    def emb(cat_t_hbm, tab_t_hbm, out_hbm, idx_v, tab_v, out_v, sem_in, sem_out):
        w = lax.axis_index("s") * NC + lax.axis_index("c")
        b0 = w * BBLK
        t0 = w * NT

        def load_unit(i, slot):
            cp_i = pltpu.async_copy(
                cat_t_hbm.at[i, pl.ds(b0, BBLK)], idx_v.at[slot], sem_in
            )
            cp_t = pltpu.async_copy(tab_t_hbm.at[i], tab_v.at[slot], sem_in)
            return cp_i, cp_t

        def compute_unit(i, slot):
            phi = (DIM * i) % 8

            @plsc.parallel_loop(0, NV)
            def vec_body(s):
                t, sl = s // 8, s % 8
                iv = idx_v[slot, pl.ds(s * L, L)]
                for j0 in range(0, DIM, GB):
                    vals = [
                        plsc.load_gather(tab_v.at[slot], [iv + jl * VOCAB])
                        for jl in range(j0, j0 + GB)
                    ]
                    for jl, v in zip(range(j0, j0 + GB), vals):
                        al, sub = (phi + jl) // 8, (phi + jl) % 8
                        out_v[slot * 8 + al, t, sub, pl.ds(sl * L, L)] = v

        def store_unit(i, slot):
            phi = (DIM * i) % 8
            a0 = (DIM * i) // 8
            span = (phi + DIM + 7) // 8
            copies = []
            for al in range(span):
                r0 = phi if al == 0 else 0
                r1 = phi + DIM - 8 * al
                r1 = 8 if r1 > 8 else r1
                rn = r1 - r0
                copies.append(
                    pltpu.async_copy(
                        out_v.at[slot * 8 + al, :, pl.ds(r0, rn), :],
                        out_hbm.at[a0 + al, pl.ds(t0, NT), pl.ds(r0, rn), :],
                        sem_out,
                    )
                )
            return copies

        loads = load_unit(0, 0)
        stores = [None, None]
        for i in range(N_COLS):
            slot = i % 2
            for cp in loads:
                cp.wait()
            if i + 1 < N_COLS:
                loads = load_unit(i + 1, (i + 1) % 2)
            if stores[slot] is not None:
                for cp in stores[slot]:
                    cp.wait()
                stores[slot] = None
            compute_unit(i, slot)
            stores[slot] = store_unit(i, slot)
        for st in stores:
            if st is not None:
                for cp in st:
                    cp.wait()

    return emb


_emb = _make_kernel()


def kernel(cat_tensor, tables):
    cat_t = cat_tensor.T
    tab_t = tables.transpose(0, 2, 1).reshape(N_COLS, DIM * VOCAB)
    out4 = _emb(cat_t, tab_t)
    out = out4.transpose(0, 2, 1, 3).reshape(OUTC_PAD, BATCH).T
    return out[:, :OUTC]
